# baseline (speedup 1.0000x reference)
import functools
import math

import jax
import jax.numpy as jnp
from jax import lax
from jax.experimental import pallas as pl
from jax.experimental.pallas import tpu as pltpu
from jax.experimental.pallas import tpu_sc as plsc

D_M = 1024
NC, NS, L = 2, 16, 16
NW = NC * NS
B_TOT = 4 * 2048
B_PER_W = B_TOT // NW
C = 32
H = C // 2
NCHUNK = B_PER_W // C
SCALE = math.sqrt(float(D_M))

_mesh = plsc.VectorSubcoreMesh(
    core_axis_name="c", subcore_axis_name="s", num_cores=NC, num_subcores=NS
)


@functools.partial(
    pl.kernel,
    out_type=jax.ShapeDtypeStruct((B_TOT, D_M), jnp.float32),
    mesh=_mesh,
    scratch_types=[
        pltpu.VMEM((NCHUNK, C), jnp.int32),
        [pltpu.VMEM((C, D_M), jnp.float32)] * 2,
        [pltpu.VMEM((H, D_M), jnp.float32)] * 2,
        [pltpu.SemaphoreType.DMA] * 2,
        [pltpu.SemaphoreType.DMA] * 2,
    ],
)
def _emb_lookup(x_hbm, lut_hbm, out_hbm, idx_v, gbufs, sbufs, gsems, ssems):
    wid = lax.axis_index("s") * NC + lax.axis_index("c")
    base = wid * B_PER_W

    pltpu.sync_copy(x_hbm.at[wid], idx_v)

    def start_gather(g):
        b = g & 1
        return pltpu.async_copy(lut_hbm.at[idx_v.at[g]], gbufs[b], gsems[b])

    def scale_half(src, h):
        dst = sbufs[h]

        @pl.loop(0, H, unroll=1)
        def _rows(r, src=src, dst=dst, h=h):
            @pl.loop(0, D_M // L, unroll=8)
            def _cols(cidx, r=r, src=src, dst=dst, h=h):
                sl = pl.ds(cidx * L, L)
                dst[r, sl] = src[r + h * H, sl] * SCALE

        return dst

    gd = [None] * NCHUNK
    sd = [None, None]
    gd[0] = start_gather(0)
    gd[1] = start_gather(1)
    for g in range(NCHUNK):
        b = g & 1
        gd[g].wait()
        src = gbufs[b]
        for h in (0, 1):
            if sd[h] is not None:
                sd[h].wait()
            dst = scale_half(src, h)
            sd[h] = pltpu.async_copy(
                dst, out_hbm.at[pl.ds(base + g * C + h * H, H)], ssems[h]
            )
        if g + 2 < NCHUNK:
            gd[g + 2] = start_gather(g + 2)
    sd[0].wait()
    sd[1].wait()


def kernel(x, lut):
    xr = x.reshape(NW, NCHUNK, C).astype(jnp.int32)
    out = _emb_lookup(xr, lut)
    return out.reshape(x.shape + (lut.shape[1],))


# speedup vs baseline: 2.3432x; 2.3432x over previous
import functools
import math

import jax
import jax.numpy as jnp
from jax import lax
from jax.experimental import pallas as pl
from jax.experimental.pallas import tpu as pltpu
from jax.experimental.pallas import tpu_sc as plsc

D_M = 1024
NC, NS, L = 2, 16, 16
NW = NC * NS
B_TOT = 4 * 2048
B_PER_W = B_TOT // NW
C = 32
NCHUNK = B_PER_W // C
NBUF = 3
SCALE = math.sqrt(float(D_M))

_mesh = plsc.VectorSubcoreMesh(
    core_axis_name="c", subcore_axis_name="s", num_cores=NC, num_subcores=NS
)


@functools.partial(
    pl.kernel,
    out_type=jax.ShapeDtypeStruct((B_TOT, D_M), jnp.float32),
    mesh=_mesh,
    scratch_types=[
        pltpu.VMEM((NCHUNK, C), jnp.int32),
        [pltpu.VMEM((C, D_M), jnp.float32)] * NBUF,
        [pltpu.SemaphoreType.DMA] * NBUF,
        [pltpu.SemaphoreType.DMA] * NBUF,
    ],
)
def _emb_lookup(x_hbm, lut_hbm, out_hbm, idx_v, bufs, gsems, ssems):
    wid = lax.axis_index("s") * NC + lax.axis_index("c")
    base = wid * B_PER_W

    pltpu.sync_copy(x_hbm.at[wid], idx_v)

    def start_gather(g):
        b = g % NBUF
        return pltpu.async_copy(lut_hbm.at[idx_v.at[g]], bufs[b], gsems[b])

    gd = [None] * NCHUNK
    sd = [None] * NCHUNK
    gd[0] = start_gather(0)
    gd[1] = start_gather(1)
    for g in range(NCHUNK):
        b = g % NBUF
        gd[g].wait()
        buf = bufs[b]

        @pl.loop(0, C, unroll=1)
        def _rows(r, buf=buf):
            @pl.loop(0, D_M // L, unroll=8)
            def _cols(cidx, r=r, buf=buf):
                sl = pl.ds(cidx * L, L)
                buf[r, sl] = buf[r, sl] * SCALE

        sd[g] = pltpu.async_copy(
            buf, out_hbm.at[pl.ds(base + g * C, C)], ssems[b]
        )
        if g + 2 < NCHUNK:
            if g >= 1:
                sd[g - 1].wait()
            gd[g + 2] = start_gather(g + 2)
    for g in range(NCHUNK - 3, NCHUNK):
        if sd[g] is not None:
            sd[g].wait()


def kernel(x, lut):
    xr = x.reshape(NW, NCHUNK, C).astype(jnp.int32)
    out = _emb_lookup(xr, lut)
    return out.reshape(x.shape + (lut.shape[1],))


# speedup vs baseline: 2.3597x; 1.0070x over previous
import functools
import math

import jax
import jax.numpy as jnp
from jax import lax
from jax.experimental import pallas as pl
from jax.experimental.pallas import tpu as pltpu
from jax.experimental.pallas import tpu_sc as plsc

D_M = 1024
NC, NS, L = 2, 16, 16
NW = NC * NS
XROWS, XCOLS = 4, 2048
B_PER_W = XROWS * XCOLS // NW
C = 32
NCHUNK = B_PER_W // C
SPANS = XCOLS // B_PER_W
SCALE = math.sqrt(float(D_M))

_mesh = plsc.VectorSubcoreMesh(
    core_axis_name="c", subcore_axis_name="s", num_cores=NC, num_subcores=NS
)


@functools.partial(
    pl.kernel,
    out_type=jax.ShapeDtypeStruct((XROWS, XCOLS, D_M), jnp.float32),
    mesh=_mesh,
    scratch_types=[
        pltpu.VMEM((B_PER_W,), jnp.int32),
        pltpu.VMEM((C, D_M), jnp.float32),
        pltpu.VMEM((C, D_M), jnp.float32),
        pltpu.SemaphoreType.DMA,
        pltpu.SemaphoreType.DMA,
        pltpu.SemaphoreType.DMA,
        pltpu.SemaphoreType.DMA,
    ],
)
def _emb_lookup(x_hbm, lut_hbm, out_hbm, idx_v, buf0, buf1,
                gsem0, gsem1, ssem0, ssem1):
    wid = lax.axis_index("s") * NC + lax.axis_index("c")
    xrow = wid // SPANS
    xcol = (wid % SPANS) * B_PER_W
    bufs = (buf0, buf1)
    gsems = (gsem0, gsem1)
    ssems = (ssem0, ssem1)

    pltpu.sync_copy(x_hbm.at[xrow, pl.ds(xcol, B_PER_W)], idx_v)

    def start_gather(g):
        b = g & 1
        return pltpu.async_copy(
            lut_hbm.at[idx_v.at[pl.ds(g * C, C)]], bufs[b], gsems[b]
        )

    gd = [None] * NCHUNK
    sd = [None] * NCHUNK
    gd[0] = start_gather(0)
    for g in range(NCHUNK):
        b = g & 1
        gd[g].wait()
        if g + 1 < NCHUNK:
            if g >= 1:
                sd[g - 1].wait()
            gd[g + 1] = start_gather(g + 1)
        buf = bufs[b]

        @pl.loop(0, C, unroll=1)
        def _rows(r, buf=buf):
            @pl.loop(0, D_M // L, unroll=8)
            def _cols(cidx, r=r, buf=buf):
                sl = pl.ds(cidx * L, L)
                buf[r, sl] = buf[r, sl] * SCALE

        sd[g] = pltpu.async_copy(
            buf, out_hbm.at[xrow, pl.ds(xcol + g * C, C)], ssems[b]
        )
    sd[NCHUNK - 2].wait()
    sd[NCHUNK - 1].wait()


def kernel(x, lut):
    return _emb_lookup(x.astype(jnp.int32), lut)


# speedup vs baseline: 2.3603x; 1.0003x over previous
import functools
import math

import jax
import jax.numpy as jnp
from jax import lax
from jax.experimental import pallas as pl
from jax.experimental.pallas import tpu as pltpu
from jax.experimental.pallas import tpu_sc as plsc

D_M = 1024
NC, NS, L = 2, 16, 16
NW = NC * NS
XROWS, XCOLS = 4, 2048
B_PER_W = XROWS * XCOLS // NW
C = 32
NCHUNK = B_PER_W // C
SPANS = XCOLS // B_PER_W
SCALE = math.sqrt(float(D_M))

_mesh = plsc.VectorSubcoreMesh(
    core_axis_name="c", subcore_axis_name="s", num_cores=NC, num_subcores=NS
)


@functools.partial(
    pl.kernel,
    out_type=jax.ShapeDtypeStruct((XROWS, XCOLS, D_M), jnp.float32),
    mesh=_mesh,
    scratch_types=[
        pltpu.VMEM((B_PER_W,), jnp.int32),
        pltpu.VMEM((C, D_M), jnp.float32),
        pltpu.VMEM((C, D_M), jnp.float32),
        pltpu.SemaphoreType.DMA,
        pltpu.SemaphoreType.DMA,
        pltpu.SemaphoreType.DMA,
        pltpu.SemaphoreType.DMA,
    ],
)
def _emb_lookup(x_hbm, lut_hbm, out_hbm, idx_v, buf0, buf1,
                gsem0, gsem1, ssem0, ssem1):
    wid = lax.axis_index("s") * NC + lax.axis_index("c")
    xrow = wid // SPANS
    xcol = (wid % SPANS) * B_PER_W
    bufs = (buf0, buf1)
    gsems = (gsem0, gsem1)
    ssems = (ssem0, ssem1)

    HALF = B_PER_W // 2
    pltpu.sync_copy(x_hbm.at[xrow, pl.ds(xcol, HALF)], idx_v.at[pl.ds(0, HALF)])

    def start_gather(g):
        b = g & 1
        return pltpu.async_copy(
            lut_hbm.at[idx_v.at[pl.ds(g * C, C)]], bufs[b], gsems[b]
        )

    gd = [None] * NCHUNK
    sd = [None] * NCHUNK
    gd[0] = start_gather(0)
    pltpu.sync_copy(
        x_hbm.at[xrow, pl.ds(xcol + HALF, HALF)],
        idx_v.at[pl.ds(HALF, HALF)],
    )
    for g in range(NCHUNK):
        b = g & 1
        gd[g].wait()
        if g + 1 < NCHUNK:
            if g >= 1:
                sd[g - 1].wait()
            gd[g + 1] = start_gather(g + 1)
        buf = bufs[b]

        @pl.loop(0, C, unroll=1)
        def _rows(r, buf=buf):
            @pl.loop(0, D_M // L, unroll=8)
            def _cols(cidx, r=r, buf=buf):
                sl = pl.ds(cidx * L, L)
                buf[r, sl] = buf[r, sl] * SCALE

        sd[g] = pltpu.async_copy(
            buf, out_hbm.at[xrow, pl.ds(xcol + g * C, C)], ssems[b]
        )
    sd[NCHUNK - 2].wait()
    sd[NCHUNK - 1].wait()


def kernel(x, lut):
    return _emb_lookup(x.astype(jnp.int32), lut)
